# baseline (speedup 1.0000x reference)
import functools

import jax
import jax.numpy as jnp
from jax import lax
from jax.experimental import pallas as pl
from jax.experimental.pallas import tpu as pltpu
from jax.experimental.pallas import tpu_sc as plsc

VOCAB = 32000
PAD = 0
SMOOTH = 0.1
CONF = 1.0 - SMOOTH
EPS = SMOOTH / (VOCAB - 2)

ROWS = 256
VB = 6400

NC = 2
NS = 16
NW = NC * NS
LANES = 16


def _sc_gather_body(pred_hbm, tgt_hbm, out_hbm, tgt_v, idx_v, val_v, sem,
                    per_w):
    wid = lax.axis_index("s") * NC + lax.axis_index("c")
    base = wid * per_w
    pltpu.sync_copy(tgt_hbm.at[pl.ds(base, per_w)], tgt_v)
    for k in range(per_w // LANES):
        t = tgt_v[pl.ds(k * LANES, LANES)]
        rows = base + k * LANES + lax.iota(jnp.int32, LANES)
        idx_v[pl.ds(k * LANES, LANES)] = rows * VOCAB + t
    pltpu.async_copy(pred_hbm.at[idx_v], val_v, sem).wait()
    pltpu.sync_copy(val_v, out_hbm.at[pl.ds(base, per_w)])


def _sc_gather(pred_flat, tgt, n):
    per_w = n // NW
    mesh = plsc.VectorSubcoreMesh(core_axis_name="c", subcore_axis_name="s")
    return pl.kernel(
        functools.partial(_sc_gather_body, per_w=per_w),
        out_type=jax.ShapeDtypeStruct((n,), jnp.float32),
        mesh=mesh,
        scratch_types=[
            pltpu.VMEM((per_w,), jnp.int32),
            pltpu.VMEM((per_w,), jnp.int32),
            pltpu.VMEM((per_w,), jnp.float32),
            pltpu.SemaphoreType.DMA,
        ],
    )(pred_flat, tgt)


def _tc_body(tgt_ref, ptgt_ref, x_ref, out_ref, m_ref, s_ref, sum_ref,
             p0_ref, acc_ref):
    i = pl.program_id(0)
    j = pl.program_id(1)
    ni = pl.num_programs(0)
    nj = pl.num_programs(1)
    x = x_ref[...]

    @pl.when(j == 0)
    def _init():
        m_ref[...] = jnp.full((ROWS, 1), -jnp.inf, jnp.float32)
        s_ref[...] = jnp.zeros((ROWS, 1), jnp.float32)
        sum_ref[...] = jnp.zeros((ROWS, 1), jnp.float32)
        p0_ref[...] = x[:, 0:1]

    @pl.when((i == 0) & (j == 0))
    def _init_acc():
        acc_ref[0] = 0.0

    m_old = m_ref[...]
    m_new = jnp.maximum(m_old, jnp.max(x, axis=1, keepdims=True))
    s_ref[...] = (s_ref[...] * jnp.exp(m_old - m_new)
                  + jnp.sum(jnp.exp(x - m_new), axis=1, keepdims=True))
    m_ref[...] = m_new
    sum_ref[...] += jnp.sum(x, axis=1, keepdims=True)

    @pl.when(j == nj - 1)
    def _fin():
        tgt = tgt_ref[...]
        lse = m_ref[...] + jnp.log(s_ref[...])
        loss = (lse - EPS * (sum_ref[...] - p0_ref[...])
                - (CONF - EPS) * ptgt_ref[...])
        loss = jnp.where(tgt != PAD, loss, 0.0)
        acc_ref[0] += jnp.sum(loss)

    @pl.when((i == ni - 1) & (j == nj - 1))
    def _out():
        out_ref[0, 0] = acc_ref[0]


def kernel(pred, target):
    n = pred.shape[0] * pred.shape[1]
    pred2 = pred.reshape(n, VOCAB)
    ni = n // ROWS
    tgt1 = target.astype(jnp.int32).reshape(n)

    ptgt = _sc_gather(pred.reshape(n * VOCAB), tgt1, n)

    out = pl.pallas_call(
        _tc_body,
        grid=(ni, VOCAB // VB),
        in_specs=[
            pl.BlockSpec((ROWS, 1), lambda i, j: (i, 0)),
            pl.BlockSpec((ROWS, 1), lambda i, j: (i, 0)),
            pl.BlockSpec((ROWS, VB), lambda i, j: (i, j)),
        ],
        out_specs=pl.BlockSpec((1, 1), lambda i, j: (0, 0),
                               memory_space=pltpu.SMEM),
        out_shape=jax.ShapeDtypeStruct((1, 1), jnp.float32),
        scratch_shapes=[
            pltpu.VMEM((ROWS, 1), jnp.float32),
            pltpu.VMEM((ROWS, 1), jnp.float32),
            pltpu.VMEM((ROWS, 1), jnp.float32),
            pltpu.VMEM((ROWS, 1), jnp.float32),
            pltpu.SMEM((1,), jnp.float32),
        ],
        compiler_params=pltpu.CompilerParams(
            dimension_semantics=("arbitrary", "arbitrary")),
    )(tgt1.reshape(n, 1), ptgt.reshape(n, 1), pred2)
    return out[0, 0] / n


# speedup vs baseline: 1.0054x; 1.0054x over previous
import functools

import jax
import jax.numpy as jnp
from jax import lax
from jax.experimental import pallas as pl
from jax.experimental.pallas import tpu as pltpu
from jax.experimental.pallas import tpu_sc as plsc

VOCAB = 32000
PAD = 0
SMOOTH = 0.1
CONF = 1.0 - SMOOTH
EPS = SMOOTH / (VOCAB - 2)

ROWS = 256
VB = 6400

NC = 2
NS = 16
NW = NC * NS
LANES = 16


def _sc_body(pred_hbm, tgt_hbm, out_hbm, tgt_v, idx_v, val_v, acc_v, sem,
             per_w):
    wid = lax.axis_index("s") * NC + lax.axis_index("c")
    base = wid * per_w
    pltpu.sync_copy(tgt_hbm.at[pl.ds(base, per_w)], tgt_v)
    for k in range(per_w // LANES):
        t = tgt_v[pl.ds(k * LANES, LANES)]
        rows = base + k * LANES + lax.iota(jnp.int32, LANES)
        idx_v[pl.ds(k * LANES, LANES)] = rows * VOCAB + t
    pltpu.async_copy(pred_hbm.at[idx_v], val_v, sem).wait()
    acc = jnp.zeros((LANES,), jnp.float32)
    for k in range(per_w // LANES):
        t = tgt_v[pl.ds(k * LANES, LANES)]
        v = val_v[pl.ds(k * LANES, LANES)]
        acc = acc + jnp.where(t != PAD, v, 0.0)
    acc_v[...] = acc
    pltpu.sync_copy(acc_v, out_hbm.at[wid])


def _sc_tgt_partials(pred_flat, tgt, n):
    per_w = n // NW
    mesh = plsc.VectorSubcoreMesh(core_axis_name="c", subcore_axis_name="s")
    return pl.kernel(
        functools.partial(_sc_body, per_w=per_w),
        out_type=jax.ShapeDtypeStruct((NW, LANES), jnp.float32),
        mesh=mesh,
        scratch_types=[
            pltpu.VMEM((per_w,), jnp.int32),
            pltpu.VMEM((per_w,), jnp.int32),
            pltpu.VMEM((per_w,), jnp.float32),
            pltpu.VMEM((LANES,), jnp.float32),
            pltpu.SemaphoreType.DMA,
        ],
    )(pred_flat, tgt)


def _tc_body(tgt_ref, x_ref, out_ref, m_ref, s_ref, sum_ref, p0_ref,
             acc_ref):
    i = pl.program_id(0)
    j = pl.program_id(1)
    ni = pl.num_programs(0)
    nj = pl.num_programs(1)
    x = x_ref[...]

    @pl.when(j == 0)
    def _init():
        m_ref[...] = jnp.full((ROWS, 1), -jnp.inf, jnp.float32)
        s_ref[...] = jnp.zeros((ROWS, 1), jnp.float32)
        sum_ref[...] = jnp.zeros((ROWS, 1), jnp.float32)
        p0_ref[...] = x[:, 0:1]

    @pl.when((i == 0) & (j == 0))
    def _init_acc():
        acc_ref[0] = 0.0

    m_old = m_ref[...]
    m_new = jnp.maximum(m_old, jnp.max(x, axis=1, keepdims=True))
    s_ref[...] = (s_ref[...] * jnp.exp(m_old - m_new)
                  + jnp.sum(jnp.exp(x - m_new), axis=1, keepdims=True))
    m_ref[...] = m_new
    sum_ref[...] += jnp.sum(x, axis=1, keepdims=True)

    @pl.when(j == nj - 1)
    def _fin():
        tgt = tgt_ref[...]
        lse = m_ref[...] + jnp.log(s_ref[...])
        part = lse - EPS * (sum_ref[...] - p0_ref[...])
        part = jnp.where(tgt != PAD, part, 0.0)
        acc_ref[0] += jnp.sum(part)

    @pl.when((i == ni - 1) & (j == nj - 1))
    def _out():
        out_ref[0, 0] = acc_ref[0]


def kernel(pred, target):
    n = pred.shape[0] * pred.shape[1]
    pred2 = pred.reshape(n, VOCAB)
    ni = n // ROWS
    tgt1 = target.astype(jnp.int32).reshape(n)

    tgt_partials = _sc_tgt_partials(pred.reshape(n * VOCAB), tgt1, n)

    tc_sum = pl.pallas_call(
        _tc_body,
        grid=(ni, VOCAB // VB),
        in_specs=[
            pl.BlockSpec((ROWS, 1), lambda i, j: (i, 0)),
            pl.BlockSpec((ROWS, VB), lambda i, j: (i, j)),
        ],
        out_specs=pl.BlockSpec((1, 1), lambda i, j: (0, 0),
                               memory_space=pltpu.SMEM),
        out_shape=jax.ShapeDtypeStruct((1, 1), jnp.float32),
        scratch_shapes=[
            pltpu.VMEM((ROWS, 1), jnp.float32),
            pltpu.VMEM((ROWS, 1), jnp.float32),
            pltpu.VMEM((ROWS, 1), jnp.float32),
            pltpu.VMEM((ROWS, 1), jnp.float32),
            pltpu.SMEM((1,), jnp.float32),
        ],
        compiler_params=pltpu.CompilerParams(
            dimension_semantics=("arbitrary", "arbitrary")),
    )(tgt1.reshape(n, 1), pred2)

    total = tc_sum[0, 0] - (CONF - EPS) * jnp.sum(tgt_partials)
    return total / n


# speedup vs baseline: 2.4599x; 2.4468x over previous
import jax
import jax.numpy as jnp
from jax.experimental import pallas as pl
from jax.experimental.pallas import tpu as pltpu

VOCAB = 32000
PAD = 0
SMOOTH = 0.1
CONF = 1.0 - SMOOTH
EPS = SMOOTH / (VOCAB - 2)

ROWS = 512
VB = 6400


def _body(tgt_ref, x_ref, out_ref, m_ref, s_ref, sum_ref, tv_ref, p0_ref,
          acc_ref):
    i = pl.program_id(0)
    j = pl.program_id(1)
    ni = pl.num_programs(0)
    nj = pl.num_programs(1)
    x = x_ref[...]

    @pl.when(j == 0)
    def _init():
        m_ref[...] = jnp.full((ROWS, 1), -jnp.inf, jnp.float32)
        s_ref[...] = jnp.zeros((ROWS, 1), jnp.float32)
        sum_ref[...] = jnp.zeros((ROWS, 1), jnp.float32)
        tv_ref[...] = jnp.zeros((ROWS, 1), jnp.float32)
        p0_ref[...] = x[:, 0:1]

    @pl.when((i == 0) & (j == 0))
    def _init_acc():
        acc_ref[0] = 0.0

    m_old = m_ref[...]
    m_new = jnp.maximum(m_old, jnp.max(x, axis=1, keepdims=True))
    s_ref[...] = (s_ref[...] * jnp.exp(m_old - m_new)
                  + jnp.sum(jnp.exp(x - m_new), axis=1, keepdims=True))
    m_ref[...] = m_new
    sum_ref[...] += jnp.sum(x, axis=1, keepdims=True)

    tgt = tgt_ref[...]
    col = j * VB + jax.lax.broadcasted_iota(jnp.int32, (ROWS, VB), 1)
    hit = col == tgt
    tv_ref[...] += jnp.sum(jnp.where(hit, x, 0.0), axis=1, keepdims=True)

    @pl.when(j == nj - 1)
    def _fin():
        lse = m_ref[...] + jnp.log(s_ref[...])
        loss = (lse - EPS * (sum_ref[...] - p0_ref[...])
                - (CONF - EPS) * tv_ref[...])
        loss = jnp.where(tgt != PAD, loss, 0.0)
        acc_ref[0] += jnp.sum(loss)

    @pl.when((i == ni - 1) & (j == nj - 1))
    def _out():
        out_ref[0, 0] = acc_ref[0]


def kernel(pred, target):
    n = pred.shape[0] * pred.shape[1]
    pred2 = pred.reshape(n, VOCAB)
    ni = n // ROWS
    tgt = target.astype(jnp.int32).reshape(n, 1)

    out = pl.pallas_call(
        _body,
        grid=(ni, VOCAB // VB),
        in_specs=[
            pl.BlockSpec((ROWS, 1), lambda i, j: (i, 0)),
            pl.BlockSpec((ROWS, VB), lambda i, j: (i, j)),
        ],
        out_specs=pl.BlockSpec((1, 1), lambda i, j: (0, 0),
                               memory_space=pltpu.SMEM),
        out_shape=jax.ShapeDtypeStruct((1, 1), jnp.float32),
        scratch_shapes=[
            pltpu.VMEM((ROWS, 1), jnp.float32),
            pltpu.VMEM((ROWS, 1), jnp.float32),
            pltpu.VMEM((ROWS, 1), jnp.float32),
            pltpu.VMEM((ROWS, 1), jnp.float32),
            pltpu.VMEM((ROWS, 1), jnp.float32),
            pltpu.SMEM((1,), jnp.float32),
        ],
        compiler_params=pltpu.CompilerParams(
            dimension_semantics=("arbitrary", "arbitrary")),
    )(tgt, pred2)
    return out[0, 0] / n


# speedup vs baseline: 2.6172x; 1.0639x over previous
import jax
import jax.numpy as jnp
from jax.experimental import pallas as pl
from jax.experimental.pallas import tpu as pltpu

VOCAB = 32000
PAD = 0
SMOOTH = 0.1
CONF = 1.0 - SMOOTH
EPS = SMOOTH / (VOCAB - 2)

ROWS = 256
VB = 16000


def _body(tgt_ref, x_ref, out_ref, m_ref, s_ref, sum_ref, tv_ref, p0_ref,
          acc_ref):
    i = pl.program_id(0)
    j = pl.program_id(1)
    ni = pl.num_programs(0)
    nj = pl.num_programs(1)
    x = x_ref[...]

    @pl.when(j == 0)
    def _init():
        m_ref[...] = jnp.full((ROWS, 1), -jnp.inf, jnp.float32)
        s_ref[...] = jnp.zeros((ROWS, 1), jnp.float32)
        sum_ref[...] = jnp.zeros((ROWS, 1), jnp.float32)
        tv_ref[...] = jnp.zeros((ROWS, 1), jnp.float32)
        p0_ref[...] = x[:, 0:1]

    @pl.when((i == 0) & (j == 0))
    def _init_acc():
        acc_ref[0] = 0.0

    m_old = m_ref[...]
    m_new = jnp.maximum(m_old, jnp.max(x, axis=1, keepdims=True))
    s_ref[...] = (s_ref[...] * jnp.exp(m_old - m_new)
                  + jnp.sum(jnp.exp(x - m_new), axis=1, keepdims=True))
    m_ref[...] = m_new
    sum_ref[...] += jnp.sum(x, axis=1, keepdims=True)

    tgt = tgt_ref[...]
    col = j * VB + jax.lax.broadcasted_iota(jnp.int32, (ROWS, VB), 1)
    hit = col == tgt
    tv_ref[...] += jnp.sum(jnp.where(hit, x, 0.0), axis=1, keepdims=True)

    @pl.when(j == nj - 1)
    def _fin():
        lse = m_ref[...] + jnp.log(s_ref[...])
        loss = (lse - EPS * (sum_ref[...] - p0_ref[...])
                - (CONF - EPS) * tv_ref[...])
        loss = jnp.where(tgt != PAD, loss, 0.0)
        acc_ref[0] += jnp.sum(loss)

    @pl.when((i == ni - 1) & (j == nj - 1))
    def _out():
        out_ref[0, 0] = acc_ref[0]


def kernel(pred, target):
    n = pred.shape[0] * pred.shape[1]
    pred2 = pred.reshape(n, VOCAB)
    ni = n // ROWS
    tgt = target.astype(jnp.int32).reshape(n, 1)

    out = pl.pallas_call(
        _body,
        grid=(ni, VOCAB // VB),
        in_specs=[
            pl.BlockSpec((ROWS, 1), lambda i, j: (i, 0)),
            pl.BlockSpec((ROWS, VB), lambda i, j: (i, j)),
        ],
        out_specs=pl.BlockSpec((1, 1), lambda i, j: (0, 0),
                               memory_space=pltpu.SMEM),
        out_shape=jax.ShapeDtypeStruct((1, 1), jnp.float32),
        scratch_shapes=[
            pltpu.VMEM((ROWS, 1), jnp.float32),
            pltpu.VMEM((ROWS, 1), jnp.float32),
            pltpu.VMEM((ROWS, 1), jnp.float32),
            pltpu.VMEM((ROWS, 1), jnp.float32),
            pltpu.VMEM((ROWS, 1), jnp.float32),
            pltpu.SMEM((1,), jnp.float32),
        ],
        compiler_params=pltpu.CompilerParams(
            dimension_semantics=("arbitrary", "arbitrary")),
    )(tgt, pred2)
    return out[0, 0] / n


# speedup vs baseline: 2.8437x; 1.0865x over previous
import jax
import jax.numpy as jnp
from jax.experimental import pallas as pl
from jax.experimental.pallas import tpu as pltpu

VOCAB = 32000
PAD = 0
SMOOTH = 0.1
CONF = 1.0 - SMOOTH
EPS = SMOOTH / (VOCAB - 2)

ROWS = 128


def _body(tgt_ref, x_ref, out_ref, acc_ref):
    i = pl.program_id(0)
    ni = pl.num_programs(0)
    x = x_ref[...]

    @pl.when(i == 0)
    def _init_acc():
        acc_ref[0] = 0.0

    m = jnp.max(x, axis=1, keepdims=True)
    s = jnp.sum(jnp.exp(x - m), axis=1, keepdims=True)
    sumx = jnp.sum(x, axis=1, keepdims=True)

    tgt = tgt_ref[...]
    col = jax.lax.broadcasted_iota(jnp.int32, (ROWS, VOCAB), 1)
    tv = jnp.sum(jnp.where(col == tgt, x, 0.0), axis=1, keepdims=True)

    lse = m + jnp.log(s)
    loss = lse - EPS * (sumx - x[:, 0:1]) - (CONF - EPS) * tv
    loss = jnp.where(tgt != PAD, loss, 0.0)
    acc_ref[0] += jnp.sum(loss)

    @pl.when(i == ni - 1)
    def _out():
        out_ref[0, 0] = acc_ref[0]


def kernel(pred, target):
    n = pred.shape[0] * pred.shape[1]
    pred2 = pred.reshape(n, VOCAB)
    ni = n // ROWS
    tgt = target.astype(jnp.int32).reshape(n, 1)

    out = pl.pallas_call(
        _body,
        grid=(ni,),
        in_specs=[
            pl.BlockSpec((ROWS, 1), lambda i: (i, 0)),
            pl.BlockSpec((ROWS, VOCAB), lambda i: (i, 0)),
        ],
        out_specs=pl.BlockSpec((1, 1), lambda i: (0, 0),
                               memory_space=pltpu.SMEM),
        out_shape=jax.ShapeDtypeStruct((1, 1), jnp.float32),
        scratch_shapes=[
            pltpu.SMEM((1,), jnp.float32),
        ],
        compiler_params=pltpu.CompilerParams(
            dimension_semantics=("arbitrary",)),
    )(tgt, pred2)
    return out[0, 0] / n
